# baseline (speedup 1.0000x reference)
import functools

import jax
import jax.numpy as jnp
from jax import lax
from jax.experimental import pallas as pl
from jax.experimental.pallas import tpu as pltpu
from jax.experimental.pallas import tpu_sc as plsc

N = 10000
CONN = 320000
DIN = 128
DH = 64
WAUG = 144
WU = 80
NC = 2
NS = 16
NWK = NC * NS
PER_W = CONN // NWK
CH = 80
NCHUNK = PER_W // CH
NB = 3
LAG = 2
NB2 = 4
N2 = 10240
RPT = N2 // NS
TBLK = 1000

_MESH = plsc.VectorSubcoreMesh(core_axis_name="c", subcore_axis_name="s")


def _ones_aux(m, w):
    return jnp.where(lax.broadcasted_iota(jnp.int32, (m, w), 1) == 0, 1.0, 0.0)


def _t1_body(nf_ref, w_ref, out_ref):
    d = jnp.dot(nf_ref[...], w_ref[...], preferred_element_type=jnp.float32)
    out_ref[...] = jnp.concatenate([d, _ones_aux(d.shape[0], WAUG - DIN)], axis=1)


def _t2_body(e0_ref, e1_ref, w_ref, out_ref):
    E = e0_ref[...] + e1_ref[...]
    bs = E[:, DIN:DIN + 1]
    binv = jnp.where(bs == 0.0, 0.0, 1.0 / bs)
    ee = jnp.dot(E[:, :DIN] * binv, w_ref[...], preferred_element_type=jnp.float32)
    out_ref[...] = jnp.concatenate([ee, _ones_aux(ee.shape[0], WAUG - DIN)], axis=1)


def _t3_body(f0_ref, f1_ref, ba_ref, outa_ref, outu_ref):
    F = f0_ref[...] + f1_ref[...]
    deg = F[:, DIN:DIN + 1]
    dinv = jnp.where(deg == 0.0, 0.0, 1.0 / deg)
    Au = F[:, :DIN] * dinv
    outa_ref[...] = Au[:, :DH] + ba_ref[...]
    outu_ref[...] = jnp.concatenate(
        [Au[:, DH:DIN], _ones_aux(Au.shape[0], WU - DH)], axis=1)


def _t4_body(g0_ref, g1_ref, out_ref):
    G = g0_ref[...] + g1_ref[...]
    cinv = 1.0 / jnp.maximum(G[:, DH:DH + 1], 1.0)
    out_ref[...] = G[:, :DH] * cinv


def _tc_call(body, out_shapes, inputs, in_specs, out_specs):
    grid = (N // TBLK,)
    return pl.pallas_call(
        body, grid=grid, in_specs=in_specs, out_specs=out_specs,
        out_shape=out_shapes)(*inputs)


def _row_spec(w):
    return pl.BlockSpec((TBLK, w), lambda i: (i, 0))


def _full_spec(a, b):
    return pl.BlockSpec((a, b), lambda i: (0, 0))


def _scatter_body(width, src_hbm, gsidx_hbm, zeros_hbm, out_hbm,
                  idxr, datar, acc_sh, isem, gsem):
    cid = lax.axis_index("c")
    sid = lax.axis_index("s")
    wid = sid * NC + cid
    r0 = sid * RPT
    pltpu.sync_copy(zeros_hbm.at[pl.ds(r0, RPT)], acc_sh.at[pl.ds(r0, RPT)])
    plsc.subcore_barrier()
    for b in range(NB):
        pltpu.async_copy(gsidx_hbm.at[wid, b], idxr.at[b], isem.at[b])

    def body(j, carry):
        jm = lax.rem(j, NB)
        pltpu.make_async_copy(
            gsidx_hbm.at[wid, j], idxr.at[jm], isem.at[jm]).wait()
        pltpu.async_copy(src_hbm.at[idxr.at[jm, 0]], datar.at[jm], gsem.at[jm])

        @pl.when(j >= LAG)
        def _():
            jp = j - LAG
            jpm = lax.rem(jp, NB)
            pltpu.make_async_copy(
                src_hbm.at[idxr.at[jpm, 0]], datar.at[jpm], gsem.at[jpm]).wait()
            pltpu.sync_copy(datar.at[jpm], acc_sh.at[idxr.at[jpm, 1]], add=True)

            @pl.when(jp + NB < NCHUNK)
            def _():
                pltpu.async_copy(
                    gsidx_hbm.at[wid, jp + NB], idxr.at[jpm], isem.at[jpm])
        return carry

    lax.fori_loop(0, NCHUNK, body, 0)
    for t in range(LAG):
        jp = NCHUNK - LAG + t
        jpm = jp % NB
        pltpu.make_async_copy(
            src_hbm.at[idxr.at[jpm, 0]], datar.at[jpm], gsem.at[jpm]).wait()
        pltpu.sync_copy(datar.at[jpm], acc_sh.at[idxr.at[jpm, 1]], add=True)
    plsc.subcore_barrier()
    pltpu.sync_copy(acc_sh.at[pl.ds(r0, RPT)], out_hbm.at[cid, pl.ds(r0, RPT)])


def _make_scatter_kernel(width):
    return pl.kernel(
        functools.partial(_scatter_body, width),
        out_type=jax.ShapeDtypeStruct((NC, N2, width), jnp.float32),
        mesh=_MESH,
        compiler_params=pltpu.CompilerParams(use_tc_tiling_on_sc=False),
        scratch_types=[
            pltpu.VMEM((NB, 2, CH), jnp.int32),
            pltpu.VMEM((NB, CH, width), jnp.float32),
            pltpu.VMEM_SHARED((N2, width), jnp.float32),
            pltpu.SemaphoreType.DMA((NB,)),
            pltpu.SemaphoreType.DMA((NB,)),
        ],
    )


def _compute_chunk(j, c, a_hbm, b_hbm, nidx_v, hidx_v,
                   ringa, ringb, w2_v, b2_v, outbuf, semsa, semsb, reissue):
    ra = ringa.at[c]
    rb = ringb.at[c]
    pltpu.make_async_copy(a_hbm.at[nidx_v.at[j]], ra, semsa.at[c]).wait()
    pltpu.make_async_copy(b_hbm.at[hidx_v.at[j]], rb, semsb.at[c]).wait()
    eidx = [jnp.arange(16, dtype=jnp.int32) + g * 16 for g in range(CH // 16)]
    accs = [b2_v[...] for _ in range(CH // 16)]
    for k in range(DH):
        w2k = w2_v[pl.ds(k * 16, 16)]
        kvec = jnp.full((16,), k, jnp.int32)
        for g in range(CH // 16):
            av = plsc.load_gather(ra, [eidx[g], kvec])
            bv = plsc.load_gather(rb, [eidx[g], kvec])
            accs[g] = accs[g] + jnp.maximum(av + bv, 0.0) * w2k
    for g in range(CH // 16):
        outbuf[pl.ds(j * CH + g * 16, 16)] = accs[g]
    if reissue:
        @pl.when(j + NB2 < NCHUNK)
        def _():
            pltpu.async_copy(a_hbm.at[nidx_v.at[j + NB2]], ra, semsa.at[c])
            pltpu.async_copy(b_hbm.at[hidx_v.at[j + NB2]], rb, semsb.at[c])


def _final_body(a_hbm, b_hbm, nidx_hbm, hidx_hbm, w2_hbm, b2_hbm, out_hbm,
                nidx_v, hidx_v, ringa, ringb, w2_v, b2_v, outbuf, semsa, semsb):
    cid = lax.axis_index("c")
    sid = lax.axis_index("s")
    wid = sid * NC + cid
    pltpu.sync_copy(nidx_hbm.at[wid], nidx_v)
    pltpu.sync_copy(hidx_hbm.at[wid], hidx_v)
    pltpu.sync_copy(w2_hbm, w2_v)
    pltpu.sync_copy(b2_hbm, b2_v)
    for c in range(NB2):
        pltpu.async_copy(a_hbm.at[nidx_v.at[c]], ringa.at[c], semsa.at[c])
        pltpu.async_copy(b_hbm.at[hidx_v.at[c]], ringb.at[c], semsb.at[c])

    args = (a_hbm, b_hbm, nidx_v, hidx_v,
            ringa, ringb, w2_v, b2_v, outbuf, semsa, semsb)

    def macro(m, carry):
        j0 = m * NB2
        for c in range(NB2):
            _compute_chunk(j0 + c, c, *args, True)
        return carry

    lax.fori_loop(0, NCHUNK // NB2, macro, 0)
    for j in range(NCHUNK - NCHUNK % NB2, NCHUNK):
        _compute_chunk(j, j % NB2, *args, False)
    pltpu.sync_copy(outbuf, out_hbm.at[pl.ds(wid * PER_W, PER_W)])


_final_kernel = pl.kernel(
    _final_body,
    out_type=jax.ShapeDtypeStruct((CONN,), jnp.float32),
    mesh=_MESH,
    compiler_params=pltpu.CompilerParams(
        use_tc_tiling_on_sc=False, needs_layout_passes=False),
    scratch_types=[
        pltpu.VMEM((NCHUNK, CH), jnp.int32),
        pltpu.VMEM((NCHUNK, CH), jnp.int32),
        pltpu.VMEM((NB2, CH, DH), jnp.float32),
        pltpu.VMEM((NB2, CH, DH), jnp.float32),
        pltpu.VMEM((DH * 16,), jnp.float32),
        pltpu.VMEM((16,), jnp.float32),
        pltpu.VMEM((PER_W,), jnp.float32),
        pltpu.SemaphoreType.DMA((NB2,)),
        pltpu.SemaphoreType.DMA((NB2,)),
    ],
)


def kernel(node_features, hyper_edge_index, enc_W, enc_b, W1, b1, W2, b2):
    nid = hyper_edge_index[0].astype(jnp.int32).reshape(NWK, NCHUNK, CH)
    hid = hyper_edge_index[1].astype(jnp.int32).reshape(NWK, NCHUNK, CH)
    ns_idx = jnp.stack([nid, hid], axis=2)
    sn_idx = jnp.stack([hid, nid], axis=2)
    W1a, W1b = W1[:DIN], W1[DIN:]
    W1cat = jnp.concatenate([W1a, W1b], axis=1)
    bias_a = (b1 + enc_b @ W1a + enc_b @ W1b).reshape(1, DH)
    zer_aug = jnp.zeros((N2, WAUG), jnp.float32)
    zer_u = jnp.zeros((N2, WU), jnp.float32)

    x_aug = _tc_call(
        _t1_body, jax.ShapeDtypeStruct((N, WAUG), jnp.float32),
        (node_features, enc_W),
        [_row_spec(DIN), _full_spec(DIN, DIN)], _row_spec(WAUG))

    e_part = _make_scatter_kernel(WAUG)(x_aug, ns_idx, zer_aug)

    ee_aug = _tc_call(
        _t2_body, jax.ShapeDtypeStruct((N, WAUG), jnp.float32),
        (e_part[0], e_part[1], W1cat),
        [_row_spec(WAUG), _row_spec(WAUG), _full_spec(DIN, DIN)],
        _row_spec(WAUG))

    f_part = _make_scatter_kernel(WAUG)(ee_aug, sn_idx, zer_aug)

    a2, u_aug = _tc_call(
        _t3_body,
        (jax.ShapeDtypeStruct((N, DH), jnp.float32),
         jax.ShapeDtypeStruct((N, WU), jnp.float32)),
        (f_part[0], f_part[1], bias_a),
        [_row_spec(WAUG), _row_spec(WAUG), _full_spec(1, DH)],
        (_row_spec(DH), _row_spec(WU)))

    g_part = _make_scatter_kernel(WU)(u_aug, ns_idx, zer_u)

    bm = _tc_call(
        _t4_body, jax.ShapeDtypeStruct((N, DH), jnp.float32),
        (g_part[0], g_part[1]),
        [_row_spec(WU), _row_spec(WU)], _row_spec(DH))

    w2mat = jnp.broadcast_to(W2.reshape(DH, 1), (DH, 16)).reshape(DH * 16)
    b2mat = jnp.broadcast_to(b2.reshape(1, 1), (1, 16)).reshape(16)
    out = _final_kernel(a2, bm, nid, hid, w2mat, b2mat)
    return out.reshape(CONN, 1)


# speedup vs baseline: 1.5816x; 1.5816x over previous
import functools

import jax
import jax.numpy as jnp
from jax import lax
from jax.experimental import pallas as pl
from jax.experimental.pallas import tpu as pltpu
from jax.experimental.pallas import tpu_sc as plsc

N = 10000
CONN = 320000
DIN = 128
DH = 64
WAUG = 144
WU = 80
NC = 2
NS = 16
NWK = NC * NS
PER_W = CONN // NWK
CH = 80
NCHUNK = PER_W // CH
NB = 3
LAG = 2
NB2 = 4
N2 = 10240
RPT = N2 // NS
TBLK = 1000

_MESH = plsc.VectorSubcoreMesh(core_axis_name="c", subcore_axis_name="s")


def _ones_aux(m, w):
    return jnp.where(lax.broadcasted_iota(jnp.int32, (m, w), 1) == 0, 1.0, 0.0)


def _t1_body(nf_ref, w_ref, out_ref):
    d = jnp.dot(nf_ref[...], w_ref[...], preferred_element_type=jnp.float32)
    out_ref[...] = jnp.concatenate([d, _ones_aux(d.shape[0], WAUG - DIN)], axis=1)


def _t2_body(e0_ref, e1_ref, w_ref, out_ref):
    E = e0_ref[...] + e1_ref[...]
    bs = E[:, DIN:DIN + 1]
    binv = jnp.where(bs == 0.0, 0.0, 1.0 / bs)
    ee = jnp.dot(E[:, :DIN] * binv, w_ref[...], preferred_element_type=jnp.float32)
    out_ref[...] = jnp.concatenate([ee, _ones_aux(ee.shape[0], WAUG - DIN)], axis=1)


def _t3_body(f0_ref, f1_ref, ba_ref, outa_ref, outu_ref):
    F = f0_ref[...] + f1_ref[...]
    deg = F[:, DIN:DIN + 1]
    dinv = jnp.where(deg == 0.0, 0.0, 1.0 / deg)
    Au = F[:, :DIN] * dinv
    outa_ref[...] = Au[:, :DH] + ba_ref[...]
    outu_ref[...] = jnp.concatenate(
        [Au[:, DH:DIN], _ones_aux(Au.shape[0], WU - DH)], axis=1)


def _t4_body(g0_ref, g1_ref, out_ref):
    G = g0_ref[...] + g1_ref[...]
    cinv = 1.0 / jnp.maximum(G[:, DH:DH + 1], 1.0)
    out_ref[...] = G[:, :DH] * cinv


def _tc_call(body, out_shapes, inputs, in_specs, out_specs):
    grid = (N // TBLK,)
    return pl.pallas_call(
        body, grid=grid, in_specs=in_specs, out_specs=out_specs,
        out_shape=out_shapes)(*inputs)


def _row_spec(w):
    return pl.BlockSpec((TBLK, w), lambda i: (i, 0))


def _full_spec(a, b):
    return pl.BlockSpec((a, b), lambda i: (0, 0))


def _scatter_body(width, src_hbm, gsidx_hbm, zeros_hbm, out_hbm,
                  idxr, datar, acc_sh, isem, gsem):
    cid = lax.axis_index("c")
    sid = lax.axis_index("s")
    wid = sid * NC + cid
    r0 = sid * RPT
    pltpu.sync_copy(zeros_hbm.at[pl.ds(r0, RPT)], acc_sh.at[pl.ds(r0, RPT)])
    plsc.subcore_barrier()
    for b in range(NB):
        pltpu.async_copy(gsidx_hbm.at[wid, b], idxr.at[b], isem.at[b])

    def body(j, carry):
        jm = lax.rem(j, NB)
        pltpu.make_async_copy(
            gsidx_hbm.at[wid, j], idxr.at[jm], isem.at[jm]).wait()
        pltpu.async_copy(src_hbm.at[idxr.at[jm, 0]], datar.at[jm], gsem.at[jm])

        @pl.when(j >= LAG)
        def _():
            jp = j - LAG
            jpm = lax.rem(jp, NB)
            pltpu.make_async_copy(
                src_hbm.at[idxr.at[jpm, 0]], datar.at[jpm], gsem.at[jpm]).wait()
            pltpu.sync_copy(datar.at[jpm], acc_sh.at[idxr.at[jpm, 1]], add=True)

            @pl.when(jp + NB < NCHUNK)
            def _():
                pltpu.async_copy(
                    gsidx_hbm.at[wid, jp + NB], idxr.at[jpm], isem.at[jpm])
        return carry

    lax.fori_loop(0, NCHUNK, body, 0)
    for t in range(LAG):
        jp = NCHUNK - LAG + t
        jpm = jp % NB
        pltpu.make_async_copy(
            src_hbm.at[idxr.at[jpm, 0]], datar.at[jpm], gsem.at[jpm]).wait()
        pltpu.sync_copy(datar.at[jpm], acc_sh.at[idxr.at[jpm, 1]], add=True)
    plsc.subcore_barrier()
    pltpu.sync_copy(acc_sh.at[pl.ds(r0, RPT)], out_hbm.at[cid, pl.ds(r0, RPT)])


def _make_scatter_kernel(width):
    return pl.kernel(
        functools.partial(_scatter_body, width),
        out_type=jax.ShapeDtypeStruct((NC, N2, width), jnp.float32),
        mesh=_MESH,
        compiler_params=pltpu.CompilerParams(use_tc_tiling_on_sc=False),
        scratch_types=[
            pltpu.VMEM((NB, 2, CH), jnp.int32),
            pltpu.VMEM((NB, CH, width), jnp.float32),
            pltpu.VMEM_SHARED((N2, width), jnp.float32),
            pltpu.SemaphoreType.DMA((NB,)),
            pltpu.SemaphoreType.DMA((NB,)),
        ],
    )


def _final_body(a_hbm, b_hbm, nidx_hbm, hidx_hbm, w2_hbm, b2_hbm, out_hbm,
                nidx_v, hidx_v, ringa, ringb, w2_v, b2_v, tbuf, outbuf,
                semsa, semsb):
    cid = lax.axis_index("c")
    sid = lax.axis_index("s")
    wid = sid * NC + cid
    pltpu.sync_copy(nidx_hbm.at[wid], nidx_v)
    pltpu.sync_copy(hidx_hbm.at[wid], hidx_v)
    pltpu.sync_copy(w2_hbm, w2_v)
    pltpu.sync_copy(b2_hbm, b2_v)
    for c in range(NB2):
        pltpu.async_copy(a_hbm.at[nidx_v.at[c]], ringa.at[c], semsa.at[c])
        pltpu.async_copy(b_hbm.at[hidx_v.at[c]], ringb.at[c], semsb.at[c])

    iot = jnp.arange(16, dtype=jnp.int32)

    def body(j, carry):
        jm = lax.rem(j, NB2)
        pltpu.make_async_copy(
            a_hbm.at[nidx_v.at[j]], ringa.at[jm], semsa.at[jm]).wait()
        pltpu.make_async_copy(
            b_hbm.at[hidx_v.at[j]], ringb.at[jm], semsb.at[jm]).wait()
        w2q = [w2_v[pl.ds(i * 16, 16)] for i in range(DH // 16)]
        b2v = b2_v[...]
        for g in range(CH // 16):
            for e in range(16):
                ee = g * 16 + e
                t = None
                for i in range(DH // 16):
                    av = ringa[jm, ee, pl.ds(i * 16, 16)]
                    bv = ringb[jm, ee, pl.ds(i * 16, 16)]
                    term = jnp.maximum(av + bv, 0.0) * w2q[i]
                    t = term if t is None else t + term
                tbuf[e, pl.ds(0, 16)] = t
            acc = b2v
            for l in range(16):
                acc = acc + plsc.load_gather(
                    tbuf, [iot, jnp.full((16,), l, jnp.int32)])
            outbuf[pl.ds(j * CH + g * 16, 16)] = acc

        @pl.when(j + NB2 < NCHUNK)
        def _():
            pltpu.async_copy(
                a_hbm.at[nidx_v.at[j + NB2]], ringa.at[jm], semsa.at[jm])
            pltpu.async_copy(
                b_hbm.at[hidx_v.at[j + NB2]], ringb.at[jm], semsb.at[jm])
        return carry

    lax.fori_loop(0, NCHUNK, body, 0)
    pltpu.sync_copy(outbuf, out_hbm.at[pl.ds(wid * PER_W, PER_W)])


_final_kernel = pl.kernel(
    _final_body,
    out_type=jax.ShapeDtypeStruct((CONN,), jnp.float32),
    mesh=_MESH,
    compiler_params=pltpu.CompilerParams(
        use_tc_tiling_on_sc=False, needs_layout_passes=False),
    scratch_types=[
        pltpu.VMEM((NCHUNK, CH), jnp.int32),
        pltpu.VMEM((NCHUNK, CH), jnp.int32),
        pltpu.VMEM((NB2, CH, DH), jnp.float32),
        pltpu.VMEM((NB2, CH, DH), jnp.float32),
        pltpu.VMEM((DH * 16,), jnp.float32),
        pltpu.VMEM((16,), jnp.float32),
        pltpu.VMEM((16, 17), jnp.float32),
        pltpu.VMEM((PER_W,), jnp.float32),
        pltpu.SemaphoreType.DMA((NB2,)),
        pltpu.SemaphoreType.DMA((NB2,)),
    ],
)


def kernel(node_features, hyper_edge_index, enc_W, enc_b, W1, b1, W2, b2):
    nid = hyper_edge_index[0].astype(jnp.int32).reshape(NWK, NCHUNK, CH)
    hid = hyper_edge_index[1].astype(jnp.int32).reshape(NWK, NCHUNK, CH)
    ns_idx = jnp.stack([nid, hid], axis=2)
    sn_idx = jnp.stack([hid, nid], axis=2)
    W1a, W1b = W1[:DIN], W1[DIN:]
    W1cat = jnp.concatenate([W1a, W1b], axis=1)
    bias_a = (b1 + enc_b @ W1a + enc_b @ W1b).reshape(1, DH)
    zer_aug = jnp.zeros((N2, WAUG), jnp.float32)
    zer_u = jnp.zeros((N2, WU), jnp.float32)

    x_aug = _tc_call(
        _t1_body, jax.ShapeDtypeStruct((N, WAUG), jnp.float32),
        (node_features, enc_W),
        [_row_spec(DIN), _full_spec(DIN, DIN)], _row_spec(WAUG))

    e_part = _make_scatter_kernel(WAUG)(x_aug, ns_idx, zer_aug)

    ee_aug = _tc_call(
        _t2_body, jax.ShapeDtypeStruct((N, WAUG), jnp.float32),
        (e_part[0], e_part[1], W1cat),
        [_row_spec(WAUG), _row_spec(WAUG), _full_spec(DIN, DIN)],
        _row_spec(WAUG))

    f_part = _make_scatter_kernel(WAUG)(ee_aug, sn_idx, zer_aug)

    a2, u_aug = _tc_call(
        _t3_body,
        (jax.ShapeDtypeStruct((N, DH), jnp.float32),
         jax.ShapeDtypeStruct((N, WU), jnp.float32)),
        (f_part[0], f_part[1], bias_a),
        [_row_spec(WAUG), _row_spec(WAUG), _full_spec(1, DH)],
        (_row_spec(DH), _row_spec(WU)))

    g_part = _make_scatter_kernel(WU)(u_aug, ns_idx, zer_u)

    bm = _tc_call(
        _t4_body, jax.ShapeDtypeStruct((N, DH), jnp.float32),
        (g_part[0], g_part[1]),
        [_row_spec(WU), _row_spec(WU)], _row_spec(DH))

    w2mat = jnp.broadcast_to(W2.reshape(DH, 1), (DH, 16)).reshape(DH * 16)
    b2mat = jnp.broadcast_to(b2.reshape(1, 1), (1, 16)).reshape(16)
    out = _final_kernel(a2, bm, nid, hid, w2mat, b2mat)
    return out.reshape(CONN, 1)
